# Initial kernel scaffold; baseline (speedup 1.0000x reference)
import jax
import jax.numpy as jnp
from jax.experimental import pallas as pl


def kernel(cls_feats, label_feats, hiddens, audio_embedding, image_results, W_ap, b_ap, Wq, bq, Wk, bk, Wv, bv, Wo, bo, ln_g, ln_b, W1, b1, W2, b2, Wfc, bfc, eW1, eb1, eW2, eb2, Wmg, bmg, Wg1, bg1, Wg2, bg2):
    raise NotImplementedError("write your pallas kernel here")


# speedup vs baseline: 4.2926x; 4.2926x over previous
import functools

import jax
import jax.numpy as jnp
from jax.experimental import pallas as pl

B = 4096
D = 768
AD = 128
H = 512
NC = 5
E = 4
MH = 128
GH = 128

BLK = 256
PAD = 128
NEG = -1e30


def _dot(a, b):
    return jnp.dot(a, b, preferred_element_type=jnp.float32)


def _fused_body(cif_ref, label_ref, aud_ref, txt_ref, noise_ref,
                W_ap_ref, b_ap_ref, Wg1_ref, bg1_ref, Wg2_ref, bg2_ref,
                Wv_ref, bv_ref, Wo_ref, bo_ref, Wmg_ref, bmg_ref,
                eW1_ref, eb1_ref, eW2_ref, eb2_ref, ln_g_ref, ln_b_ref,
                W1_ref, b1_ref, W2_ref, b2_ref, Wfc_ref, bfc_ref,
                out_ref):
    cif = cif_ref[...]
    lane = jax.lax.broadcasted_iota(jnp.int32, (BLK, PAD), 1)

    adjusted = jnp.zeros((BLK, D), jnp.float32)
    for j in range(5):
        adjusted = adjusted + jnp.where(cif == j, 1.0, 0.0) * label_ref[:, j, :]

    txt = txt_ref[...]
    x_aud = _dot(aud_ref[...], W_ap_ref[...]) + b_ap_ref[...]

    xcat = jnp.concatenate([adjusted, x_aud, txt], axis=1)
    g1 = jnp.maximum(_dot(xcat, Wg1_ref[...]) + bg1_ref[...], 0.0)
    glog = _dot(g1, Wg2_ref[...]) + bg2_ref[...]
    glm = jnp.where(lane < 3, glog, NEG)
    gmax = jnp.max(glm, axis=1, keepdims=True)
    ge = jnp.exp(glm - gmax)
    gw = ge / jnp.sum(ge, axis=1, keepdims=True)
    x_aud_s = gw[:, 1:2] * x_aud
    x_text = gw[:, 2:3] * txt

    y_att = _dot(_dot(x_aud_s, Wv_ref[...]) + bv_ref[...], Wo_ref[...]) + bo_ref[...]

    nm = _dot(y_att, Wmg_ref[...]) + bmg_ref[...] + noise_ref[...]
    nm = jnp.where(lane < E, nm, NEG)
    n0 = nm[:, 0:1]
    m1 = jnp.max(nm, axis=1, keepdims=True)
    am = jnp.min(jnp.where(nm == m1, lane, PAD), axis=1, keepdims=True)
    m2 = jnp.max(jnp.where(lane == am, NEG, nm), axis=1, keepdims=True)
    cnt = jnp.sum(jnp.where(nm > n0, 1.0, 0.0), axis=1, keepdims=True)
    s0 = jnp.where(cnt <= 1.5, jnp.exp(n0 - m1) / (1.0 + jnp.exp(m2 - m1)), 0.0)

    h = jnp.maximum(_dot(y_att, eW1_ref[...]) + eb1_ref[...], 0.0)
    y = s0 * (_dot(h, eW2_ref[...]) + eb2_ref[...])

    ln_g = ln_g_ref[...]
    ln_b = ln_b_ref[...]

    def ln(v):
        mu = jnp.mean(v, axis=1, keepdims=True)
        c = v - mu
        var = jnp.mean(c * c, axis=1, keepdims=True)
        return ln_g * c * jax.lax.rsqrt(var + 1e-5) + ln_b

    x = x_text
    for _ in range(4):
        x = ln(y + x)
        y2 = _dot(jnp.maximum(_dot(x, W1_ref[...]) + b1_ref[...], 0.0),
                  W2_ref[...]) + b2_ref[...]
        x = ln(y2 + x)

    logits = _dot(x, Wfc_ref[...]) + bfc_ref[...]
    lm = jnp.where(lane < NC, logits, NEG)
    lmax = jnp.max(lm, axis=1, keepdims=True)
    le = jnp.exp(lm - lmax)
    out_ref[...] = le / jnp.sum(le, axis=1, keepdims=True)


def _padded(w, b, cols):
    wp = jnp.zeros((w.shape[0], PAD), jnp.float32).at[:, :cols].set(w)
    bp = jnp.zeros((1, PAD), jnp.float32).at[:, :cols].set(b)
    return wp, bp


@jax.jit
def kernel(cls_feats, label_feats, hiddens, audio_embedding, image_results,
           W_ap, b_ap, Wq, bq, Wk, bk, Wv, bv, Wo, bo, ln_g, ln_b,
           W1, b1, W2, b2, Wfc, bfc, eW1, eb1, eW2, eb2, Wmg, bmg,
           Wg1, bg1, Wg2, bg2):
    del cls_feats, Wq, bq, Wk, bk

    cif = image_results.astype(jnp.float32).reshape(B, 1)
    aud = audio_embedding.reshape(B, AD)
    txt = hiddens.reshape(B, D)
    noise = (jax.random.normal(jax.random.key(1), (B, 1, E), jnp.float32)
             * 0.1).reshape(B, E)
    noise_p = jnp.zeros((B, PAD), jnp.float32).at[:, :E].set(noise)

    Wg2p, bg2p = _padded(Wg2, bg2, 3)
    Wmgp, bmgp = _padded(Wmg, bmg, E)
    Wfcp, bfcp = _padded(Wfc, bfc, NC)
    eW1c = eW1.transpose(1, 0, 2).reshape(D, E * MH)
    eb1c = eb1.reshape(1, E * MH)
    eW2c = eW2.reshape(E * MH, D)
    eb2s = eb2.sum(0).reshape(1, D)

    row2 = lambda v: v.reshape(1, -1)

    grid = (B // BLK,)
    bspec = lambda shape: pl.BlockSpec(shape, lambda i: (i, 0))
    wspec = lambda shape: pl.BlockSpec(shape, lambda i: (0, 0))

    out = pl.pallas_call(
        _fused_body,
        grid=grid,
        in_specs=[
            bspec((BLK, 1)),
            pl.BlockSpec((BLK, 6, D), lambda i: (i, 0, 0)),
            bspec((BLK, AD)),
            bspec((BLK, D)),
            bspec((BLK, PAD)),
            wspec((AD, D)), wspec((1, D)),
            wspec((3 * D, GH)), wspec((1, GH)),
            wspec((GH, PAD)), wspec((1, PAD)),
            wspec((D, D)), wspec((1, D)),
            wspec((D, D)), wspec((1, D)),
            wspec((D, PAD)), wspec((1, PAD)),
            wspec((D, E * MH)), wspec((1, E * MH)),
            wspec((E * MH, D)), wspec((1, D)),
            wspec((1, D)), wspec((1, D)),
            wspec((D, H)), wspec((1, H)),
            wspec((H, D)), wspec((1, D)),
            wspec((D, PAD)), wspec((1, PAD)),
        ],
        out_specs=bspec((BLK, PAD)),
        out_shape=jax.ShapeDtypeStruct((B, PAD), jnp.float32),
    )(cif, label_feats, aud, txt, noise_p,
      W_ap, row2(b_ap), Wg1, row2(bg1), Wg2p, bg2p,
      Wv, row2(bv), Wo, row2(bo), Wmgp, bmgp,
      eW1c, eb1c, eW2c, eb2s, row2(ln_g), row2(ln_b),
      W1, row2(b1), W2, row2(b2), Wfcp, bfcp)

    return out[:, :NC]


# speedup vs baseline: 4.4978x; 1.0478x over previous
import functools

import jax
import jax.numpy as jnp
from jax.experimental import pallas as pl

B = 4096
D = 768
AD = 128
H = 512
NC = 5
E = 4
MH = 128
GH = 128

BLK = 512
PAD = 128
NEG = -1e30


def _dot(a, b):
    return jnp.dot(a, b, preferred_element_type=jnp.float32)


def _bdot(a, b16):
    return jnp.dot(a.astype(jnp.bfloat16), b16,
                   preferred_element_type=jnp.float32)


def _fused_body(cif_ref, label_ref, aud_ref, txt_ref, noise_ref,
                W_ap_ref, b_ap_ref, Wg1_ref, bg1_ref, Wg2_ref, bg2_ref,
                Wv_ref, bv_ref, Wo_ref, bo_ref, Wmg_ref, bmg_ref,
                eW1_ref, eb1_ref, eW2_ref, eb2_ref, ln_g_ref, ln_b_ref,
                W1_ref, b1_ref, W2_ref, b2_ref, Wfc_ref, bfc_ref,
                out_ref):
    cif = cif_ref[...]
    lane = jax.lax.broadcasted_iota(jnp.int32, (BLK, PAD), 1)

    adjusted = jnp.zeros((BLK, D), jnp.float32)
    for j in range(5):
        adjusted = adjusted + jnp.where(cif == j, 1.0, 0.0) * label_ref[:, j, :]

    txt = txt_ref[...]
    x_aud = _dot(aud_ref[...], W_ap_ref[...]) + b_ap_ref[...]

    xcat = jnp.concatenate([adjusted, x_aud, txt], axis=1)
    g1 = jnp.maximum(_bdot(xcat, Wg1_ref[...]) + bg1_ref[...], 0.0)
    glog = _dot(g1, Wg2_ref[...]) + bg2_ref[...]
    glm = jnp.where(lane < 3, glog, NEG)
    gmax = jnp.max(glm, axis=1, keepdims=True)
    ge = jnp.exp(glm - gmax)
    gw = ge / jnp.sum(ge, axis=1, keepdims=True)
    x_aud_s = gw[:, 1:2] * x_aud
    x_text = gw[:, 2:3] * txt

    y_att = _bdot(_bdot(x_aud_s, Wv_ref[...]) + bv_ref[...], Wo_ref[...]) + bo_ref[...]

    nm = _dot(y_att, Wmg_ref[...]) + bmg_ref[...] + noise_ref[...]
    nm = jnp.where(lane < E, nm, NEG)
    n0 = nm[:, 0:1]
    m1 = jnp.max(nm, axis=1, keepdims=True)
    am = jnp.min(jnp.where(nm == m1, lane, PAD), axis=1, keepdims=True)
    m2 = jnp.max(jnp.where(lane == am, NEG, nm), axis=1, keepdims=True)
    cnt = jnp.sum(jnp.where(nm > n0, 1.0, 0.0), axis=1, keepdims=True)
    s0 = jnp.where(cnt <= 1.5, jnp.exp(n0 - m1) / (1.0 + jnp.exp(m2 - m1)), 0.0)

    h = jnp.maximum(_bdot(y_att, eW1_ref[...]) + eb1_ref[...], 0.0)
    y = s0 * (_bdot(h, eW2_ref[...]) + eb2_ref[...])

    ln_g = ln_g_ref[...]
    ln_b = ln_b_ref[...]

    def ln(v):
        mu = jnp.mean(v, axis=1, keepdims=True)
        c = v - mu
        var = jnp.mean(c * c, axis=1, keepdims=True)
        return ln_g * c * jax.lax.rsqrt(var + 1e-5) + ln_b

    x = x_text
    for _ in range(4):
        x = ln(y + x)
        y2 = _bdot(jnp.maximum(_bdot(x, W1_ref[...]) + b1_ref[...], 0.0),
                   W2_ref[...]) + b2_ref[...]
        x = ln(y2 + x)

    logits = _dot(x, Wfc_ref[...]) + bfc_ref[...]
    lm = jnp.where(lane < NC, logits, NEG)
    lmax = jnp.max(lm, axis=1, keepdims=True)
    le = jnp.exp(lm - lmax)
    out_ref[...] = le / jnp.sum(le, axis=1, keepdims=True)


def _padded(w, b, cols):
    wp = jnp.zeros((w.shape[0], PAD), jnp.float32).at[:, :cols].set(w)
    bp = jnp.zeros((1, PAD), jnp.float32).at[:, :cols].set(b)
    return wp, bp


@jax.jit
def kernel(cls_feats, label_feats, hiddens, audio_embedding, image_results,
           W_ap, b_ap, Wq, bq, Wk, bk, Wv, bv, Wo, bo, ln_g, ln_b,
           W1, b1, W2, b2, Wfc, bfc, eW1, eb1, eW2, eb2, Wmg, bmg,
           Wg1, bg1, Wg2, bg2):
    del cls_feats, Wq, bq, Wk, bk

    cif = image_results.astype(jnp.float32).reshape(B, 1)
    aud = audio_embedding.reshape(B, AD)
    txt = hiddens.reshape(B, D)
    noise = (jax.random.normal(jax.random.key(1), (B, 1, E), jnp.float32)
             * 0.1).reshape(B, E)
    noise_p = jnp.zeros((B, PAD), jnp.float32).at[:, :E].set(noise)

    Wg2p, bg2p = _padded(Wg2, bg2, 3)
    Wmgp, bmgp = _padded(Wmg, bmg, E)
    Wfcp, bfcp = _padded(Wfc, bfc, NC)
    bf = jnp.bfloat16
    eW1c = eW1.transpose(1, 0, 2).reshape(D, E * MH).astype(bf)
    eb1c = eb1.reshape(1, E * MH)
    eW2c = eW2.reshape(E * MH, D).astype(bf)
    eb2s = eb2.sum(0).reshape(1, D)
    Wg1h, Wvh, Woh, W1h, W2h = (w.astype(bf) for w in (Wg1, Wv, Wo, W1, W2))

    row2 = lambda v: v.reshape(1, -1)

    grid = (B // BLK,)
    bspec = lambda shape: pl.BlockSpec(shape, lambda i: (i, 0))
    wspec = lambda shape: pl.BlockSpec(shape, lambda i: (0, 0))

    out = pl.pallas_call(
        _fused_body,
        grid=grid,
        in_specs=[
            bspec((BLK, 1)),
            pl.BlockSpec((BLK, 6, D), lambda i: (i, 0, 0)),
            bspec((BLK, AD)),
            bspec((BLK, D)),
            bspec((BLK, PAD)),
            wspec((AD, D)), wspec((1, D)),
            wspec((3 * D, GH)), wspec((1, GH)),
            wspec((GH, PAD)), wspec((1, PAD)),
            wspec((D, D)), wspec((1, D)),
            wspec((D, D)), wspec((1, D)),
            wspec((D, PAD)), wspec((1, PAD)),
            wspec((D, E * MH)), wspec((1, E * MH)),
            wspec((E * MH, D)), wspec((1, D)),
            wspec((1, D)), wspec((1, D)),
            wspec((D, H)), wspec((1, H)),
            wspec((H, D)), wspec((1, D)),
            wspec((D, PAD)), wspec((1, PAD)),
        ],
        out_specs=bspec((BLK, PAD)),
        out_shape=jax.ShapeDtypeStruct((B, PAD), jnp.float32),
    )(cif, label_feats, aud, txt, noise_p,
      W_ap, row2(b_ap), Wg1h, row2(bg1), Wg2p, bg2p,
      Wvh, row2(bv), Woh, row2(bo), Wmgp, bmgp,
      eW1c, eb1c, eW2c, eb2s, row2(ln_g), row2(ln_b),
      W1h, row2(b1), W2h, row2(b2), Wfcp, bfcp)

    return out[:, :NC]
